# baseline (speedup 1.0000x reference)
import jax
import jax.numpy as jnp
from jax import lax
from jax.experimental import pallas as pl
from jax.experimental.pallas import tpu as pltpu
from jax.experimental.pallas import tpu_sc as plsc

NC = 2
NS = 16
NW = NC * NS

B = 256
ROW = 4 * 64 * 64
SPW = B // NW
LANES = 16
UNROLL = 4


def _body(x0_hbm, t_hbm, n_hbm, ac_hbm, am_hbm, out_hbm,
          xbuf, nbuf, tb_v, a_v, am_v, sem):
    wid = lax.axis_index("s") * NC + lax.axis_index("c")
    base = wid * SPW

    for s in range(SPW):
        row = base + s
        pltpu.sync_copy(x0_hbm.at[row], xbuf)
        pltpu.sync_copy(n_hbm.at[row], nbuf)

        rvec = jnp.full((LANES,), row, jnp.int32)
        pltpu.async_copy(t_hbm.at[rvec], tb_v, sem).wait()
        pltpu.async_copy(ac_hbm.at[tb_v], a_v, sem).wait()
        pltpu.async_copy(am_hbm.at[tb_v], am_v, sem).wait()
        a = a_v[...]
        am = am_v[...]

        def inner(j, carry):
            for u in range(UNROLL):
                sl = pl.ds((j * UNROLL + u) * LANES, LANES)
                xbuf[sl] = a * xbuf[sl] + am * nbuf[sl]
            return carry

        lax.fori_loop(0, ROW // (LANES * UNROLL), inner, 0)
        pltpu.sync_copy(xbuf, out_hbm.at[row])


@jax.jit
def _run(x0f, t32, nf, ac, am):
    mesh = plsc.VectorSubcoreMesh(
        core_axis_name="c", subcore_axis_name="s",
        num_cores=NC, num_subcores=NS)
    f = pl.kernel(
        _body,
        out_type=jax.ShapeDtypeStruct((B, ROW), jnp.float32),
        mesh=mesh,
        scratch_types=[
            pltpu.VMEM((ROW,), jnp.float32),
            pltpu.VMEM((ROW,), jnp.float32),
            pltpu.VMEM((LANES,), jnp.int32),
            pltpu.VMEM((LANES,), jnp.float32),
            pltpu.VMEM((LANES,), jnp.float32),
            pltpu.SemaphoreType.DMA,
        ],
    )
    return f(x0f, t32, nf, ac, am)


def kernel(x0, t, noise, sqrt_ac, sqrt_1mac):
    shape = x0.shape
    out = _run(x0.reshape(B, ROW), t.astype(jnp.int32),
               noise.reshape(B, ROW), sqrt_ac, sqrt_1mac)
    return (out.reshape(shape), noise)


# speedup vs baseline: 1.1789x; 1.1789x over previous
import jax
import jax.numpy as jnp
from jax import lax
from jax.experimental import pallas as pl
from jax.experimental.pallas import tpu as pltpu
from jax.experimental.pallas import tpu_sc as plsc

NC = 2
NS = 16
NW = NC * NS

B = 256
ROW = 4 * 64 * 64
SPW = B // NW
LANES = 16
UNROLL = 8


def _body(x0_hbm, t_hbm, n_hbm, ac_hbm, am_hbm, out_hbm,
          xb0, xb1, nb0, nb1, ob0, ob1, tb_rep, a_rep, am_rep,
          csem, isem0, isem1, osem0, osem1):
    xb = [xb0, xb1]
    nb = [nb0, nb1]
    ob = [ob0, ob1]
    isem = [isem0, isem1]
    osem = [osem0, osem1]

    wid = lax.axis_index("s") * NC + lax.axis_index("c")
    base = wid * SPW

    in_cp = [None, None]
    out_cp = [None, None]

    def start_in(s):
        p = s % 2
        in_cp[p] = (
            pltpu.async_copy(x0_hbm.at[base + s], xb[p], isem[p]),
            pltpu.async_copy(n_hbm.at[base + s], nb[p], isem[p]),
        )

    start_in(0)
    start_in(1)

    tcps = []
    for s in range(SPW):
        rvec = jnp.full((LANES,), base + s, jnp.int32)
        tcps.append(pltpu.async_copy(
            t_hbm.at[rvec], tb_rep.at[pl.ds(s * LANES, LANES)], csem))
    for cp in tcps:
        cp.wait()
    cpa = pltpu.async_copy(ac_hbm.at[tb_rep], a_rep, csem)
    cpm = pltpu.async_copy(am_hbm.at[tb_rep], am_rep, csem)
    cpa.wait()
    cpm.wait()

    for s in range(SPW):
        p = s % 2
        if out_cp[p] is not None:
            out_cp[p].wait()
        for cp in in_cp[p]:
            cp.wait()

        a = a_rep[pl.ds(s * LANES, LANES)]
        am = am_rep[pl.ds(s * LANES, LANES)]
        xbuf, nbuf, obuf = xb[p], nb[p], ob[p]

        def inner(j, carry):
            for u in range(UNROLL):
                sl = pl.ds((j * UNROLL + u) * LANES, LANES)
                obuf[sl] = a * xbuf[sl] + am * nbuf[sl]
            return carry

        lax.fori_loop(0, ROW // (LANES * UNROLL), inner, 0)

        if s + 2 < SPW:
            start_in(s + 2)
        out_cp[p] = pltpu.async_copy(obuf, out_hbm.at[base + s], osem[p])

    out_cp[0].wait()
    out_cp[1].wait()


@jax.jit
def _run(x0f, t32, nf, ac, am):
    mesh = plsc.VectorSubcoreMesh(
        core_axis_name="c", subcore_axis_name="s",
        num_cores=NC, num_subcores=NS)
    f = pl.kernel(
        _body,
        out_type=jax.ShapeDtypeStruct((B, ROW), jnp.float32),
        mesh=mesh,
        scratch_types=[
            pltpu.VMEM((ROW,), jnp.float32),
            pltpu.VMEM((ROW,), jnp.float32),
            pltpu.VMEM((ROW,), jnp.float32),
            pltpu.VMEM((ROW,), jnp.float32),
            pltpu.VMEM((ROW,), jnp.float32),
            pltpu.VMEM((ROW,), jnp.float32),
            pltpu.VMEM((SPW * LANES,), jnp.int32),
            pltpu.VMEM((SPW * LANES,), jnp.float32),
            pltpu.VMEM((SPW * LANES,), jnp.float32),
            pltpu.SemaphoreType.DMA,
            pltpu.SemaphoreType.DMA,
            pltpu.SemaphoreType.DMA,
            pltpu.SemaphoreType.DMA,
            pltpu.SemaphoreType.DMA,
        ],
    )
    return f(x0f, t32, nf, ac, am)


def kernel(x0, t, noise, sqrt_ac, sqrt_1mac):
    shape = x0.shape
    out = _run(x0.reshape(B, ROW), t.astype(jnp.int32),
               noise.reshape(B, ROW), sqrt_ac, sqrt_1mac)
    return (out.reshape(shape), noise)


# speedup vs baseline: 2.1554x; 1.8284x over previous
import jax
import jax.numpy as jnp
from jax import lax
from jax.experimental import pallas as pl
from jax.experimental.pallas import tpu as pltpu
from jax.experimental.pallas import tpu_sc as plsc

NC = 2
NS = 16
NW = NC * NS

B = 256
C, H, W = 4, 64, 64
P = C * H
PPW = P // NW
LANES = 16
BCHUNKS = B // LANES


def _body(x0_hbm, t_hbm, n_hbm, ac_hbm, am_hbm, out_hbm,
          xb0, xb1, nb0, nb1, ob0, ob1, t_v, a_all, am_all,
          csem, isem0, isem1, osem0, osem1):
    xb = [xb0, xb1]
    nb = [nb0, nb1]
    ob = [ob0, ob1]
    isem = [isem0, isem1]
    osem = [osem0, osem1]

    wid = lax.axis_index("s") * NC + lax.axis_index("c")
    base = wid * PPW

    in_cp = [None, None]
    out_cp = [None, None]

    def start_in(s):
        p = s % 2
        in_cp[p] = (
            pltpu.async_copy(x0_hbm.at[base + s], xb[p], isem[p]),
            pltpu.async_copy(n_hbm.at[base + s], nb[p], isem[p]),
        )

    start_in(0)
    start_in(1)

    pltpu.sync_copy(t_hbm, t_v)
    ccps = []
    for h in range(2):
        sl = pl.ds(h * 128, 128)
        ccps.append(pltpu.async_copy(ac_hbm.at[t_v.at[sl]], a_all.at[sl], csem))
        ccps.append(pltpu.async_copy(am_hbm.at[t_v.at[sl]], am_all.at[sl], csem))
    for cp in ccps:
        cp.wait()

    a_vecs = [a_all[pl.ds(g * LANES, LANES)] for g in range(BCHUNKS)]
    am_vecs = [am_all[pl.ds(g * LANES, LANES)] for g in range(BCHUNKS)]

    for s in range(PPW):
        p = s % 2
        if out_cp[p] is not None:
            out_cp[p].wait()
        for cp in in_cp[p]:
            cp.wait()

        xbuf, nbuf, obuf = xb[p], nb[p], ob[p]

        def inner(w, carry):
            for g in range(BCHUNKS):
                sl = pl.ds(g * LANES, LANES)
                obuf[w, sl] = (a_vecs[g] * xbuf[w, sl]
                               + am_vecs[g] * nbuf[w, sl])
            return carry

        lax.fori_loop(0, W, inner, 0)

        if s + 2 < PPW:
            start_in(s + 2)
        out_cp[p] = pltpu.async_copy(obuf, out_hbm.at[base + s], osem[p])

    out_cp[0].wait()
    out_cp[1].wait()


@jax.jit
def _run(x0, t32, noise, ac, am):
    x0T = x0.transpose(1, 2, 3, 0).reshape(P, W, B)
    nT = noise.transpose(1, 2, 3, 0).reshape(P, W, B)
    mesh = plsc.VectorSubcoreMesh(
        core_axis_name="c", subcore_axis_name="s",
        num_cores=NC, num_subcores=NS)
    f = pl.kernel(
        _body,
        out_type=jax.ShapeDtypeStruct((P, W, B), jnp.float32),
        mesh=mesh,
        compiler_params=pltpu.CompilerParams(use_tc_tiling_on_sc=True),
        scratch_types=[
            pltpu.VMEM((W, B), jnp.float32),
            pltpu.VMEM((W, B), jnp.float32),
            pltpu.VMEM((W, B), jnp.float32),
            pltpu.VMEM((W, B), jnp.float32),
            pltpu.VMEM((W, B), jnp.float32),
            pltpu.VMEM((W, B), jnp.float32),
            pltpu.VMEM((B,), jnp.int32),
            pltpu.VMEM((B,), jnp.float32),
            pltpu.VMEM((B,), jnp.float32),
            pltpu.SemaphoreType.DMA,
            pltpu.SemaphoreType.DMA,
            pltpu.SemaphoreType.DMA,
            pltpu.SemaphoreType.DMA,
            pltpu.SemaphoreType.DMA,
        ],
    )
    outT = f(x0T, t32, nT, ac, am)
    return outT.reshape(C, H, W, B).transpose(3, 0, 1, 2)


def kernel(x0, t, noise, sqrt_ac, sqrt_1mac):
    out = _run(x0, t.astype(jnp.int32), noise, sqrt_ac, sqrt_1mac)
    return (out, noise)


# speedup vs baseline: 2.3761x; 1.1024x over previous
import jax
import jax.numpy as jnp
from jax import lax
from jax.experimental import pallas as pl
from jax.experimental.pallas import tpu as pltpu
from jax.experimental.pallas import tpu_sc as plsc

NC = 2
NS = 16
NW = NC * NS

B = 256
C, H, W = 4, 64, 64
P = C * H
PPW = P // NW
LANES = 16
BCHUNKS = B // LANES


def _body(x0_hbm, t_hbm, n_hbm, ac_hbm, am_hbm, out_hbm,
          xb0, xb1, nb0, nb1, ob0, ob1, t_v, a_all, am_all,
          csem, isem0, isem1, osem0, osem1):
    xb = [xb0, xb1]
    nb = [nb0, nb1]
    ob = [ob0, ob1]
    isem = [isem0, isem1]
    osem = [osem0, osem1]

    wid = lax.axis_index("s") * NC + lax.axis_index("c")
    base = wid * PPW

    in_cp = [None, None]
    out_cp = [None, None]

    def start_in(s):
        p = s % 2
        in_cp[p] = (
            pltpu.async_copy(x0_hbm.at[base + s], xb[p], isem[p]),
            pltpu.async_copy(n_hbm.at[base + s], nb[p], isem[p]),
        )

    start_in(0)
    start_in(1)

    pltpu.sync_copy(t_hbm, t_v)
    ccps = []
    for h in range(2):
        sl = pl.ds(h * 128, 128)
        ccps.append(pltpu.async_copy(ac_hbm.at[t_v.at[sl]], a_all.at[sl], csem))
        ccps.append(pltpu.async_copy(am_hbm.at[t_v.at[sl]], am_all.at[sl], csem))
    for cp in ccps:
        cp.wait()

    a_vecs = [a_all[pl.ds(g * LANES, LANES)] for g in range(BCHUNKS)]
    am_vecs = [am_all[pl.ds(g * LANES, LANES)] for g in range(BCHUNKS)]

    for s in range(PPW):
        p = s % 2
        if out_cp[p] is not None:
            out_cp[p].wait()
        for cp in in_cp[p]:
            cp.wait()

        xbuf, nbuf, obuf = xb[p], nb[p], ob[p]

        def inner(w, carry):
            for g in range(BCHUNKS):
                sl = pl.ds(g * LANES, LANES)
                obuf[w, sl] = (a_vecs[g] * xbuf[w, sl]
                               + am_vecs[g] * nbuf[w, sl])
            return carry

        lax.fori_loop(0, W, inner, 0)

        if s + 2 < PPW:
            start_in(s + 2)
        out_cp[p] = pltpu.async_copy(obuf, out_hbm.at[base + s], osem[p])

    out_cp[0].wait()
    out_cp[1].wait()


def _tc_copy_body(n_ref, o_ref):
    o_ref[...] = n_ref[...]


def _tc_copy(nT):
    blk = 16
    return pl.pallas_call(
        _tc_copy_body,
        grid=(P // blk,),
        in_specs=[pl.BlockSpec((blk, W, B), lambda i: (i, 0, 0))],
        out_specs=pl.BlockSpec((blk, W, B), lambda i: (i, 0, 0)),
        out_shape=jax.ShapeDtypeStruct((P, W, B), jnp.float32),
    )(nT)


@jax.jit
def _run(x0, t32, noise, ac, am):
    x0T = x0.transpose(1, 2, 3, 0).reshape(P, W, B)
    nT = noise.transpose(1, 2, 3, 0).reshape(P, W, B)
    mesh = plsc.VectorSubcoreMesh(
        core_axis_name="c", subcore_axis_name="s",
        num_cores=NC, num_subcores=NS)
    f = pl.kernel(
        _body,
        out_type=jax.ShapeDtypeStruct((P, W, B), jnp.float32),
        mesh=mesh,
        compiler_params=pltpu.CompilerParams(use_tc_tiling_on_sc=True),
        scratch_types=[
            pltpu.VMEM((W, B), jnp.float32),
            pltpu.VMEM((W, B), jnp.float32),
            pltpu.VMEM((W, B), jnp.float32),
            pltpu.VMEM((W, B), jnp.float32),
            pltpu.VMEM((W, B), jnp.float32),
            pltpu.VMEM((W, B), jnp.float32),
            pltpu.VMEM((B,), jnp.int32),
            pltpu.VMEM((B,), jnp.float32),
            pltpu.VMEM((B,), jnp.float32),
            pltpu.SemaphoreType.DMA,
            pltpu.SemaphoreType.DMA,
            pltpu.SemaphoreType.DMA,
            pltpu.SemaphoreType.DMA,
            pltpu.SemaphoreType.DMA,
        ],
    )
    outT = f(x0T, t32, nT, ac, am)
    noutT = _tc_copy(nT)
    out = outT.reshape(C, H, W, B).transpose(3, 0, 1, 2)
    nout = noutT.reshape(C, H, W, B).transpose(3, 0, 1, 2)
    return out, nout


def kernel(x0, t, noise, sqrt_ac, sqrt_1mac):
    return _run(x0, t.astype(jnp.int32), noise, sqrt_ac, sqrt_1mac)


# speedup vs baseline: 2.8418x; 1.1960x over previous
import jax
import jax.numpy as jnp
from jax import lax
from jax.experimental import pallas as pl
from jax.experimental.pallas import tpu as pltpu
from jax.experimental.pallas import tpu_sc as plsc

NC = 2
NS = 16
B = 256
C, H, W = 4, 64, 64
P = C * H
PBLK = 16


def _gather_body(t_hbm, ac_hbm, am_hbm, a_out, am_out, t_v, a_v, am_v, csem):
    wid = lax.axis_index("s") * NC + lax.axis_index("c")

    @pl.when(wid == 0)
    def _():
        pltpu.sync_copy(t_hbm, t_v)
        ccps = []
        for h in range(2):
            sl = pl.ds(h * 128, 128)
            ccps.append(pltpu.async_copy(
                ac_hbm.at[t_v.at[sl]], a_v.at[sl], csem))
            ccps.append(pltpu.async_copy(
                am_hbm.at[t_v.at[sl]], am_v.at[sl], csem))
        for cp in ccps:
            cp.wait()
        pltpu.sync_copy(a_v, a_out)
        pltpu.sync_copy(am_v, am_out)


def _sc_gather(t32, ac, am):
    mesh = plsc.VectorSubcoreMesh(
        core_axis_name="c", subcore_axis_name="s",
        num_cores=NC, num_subcores=NS)
    f = pl.kernel(
        _gather_body,
        out_type=(jax.ShapeDtypeStruct((B,), jnp.float32),
                  jax.ShapeDtypeStruct((B,), jnp.float32)),
        mesh=mesh,
        scratch_types=[
            pltpu.VMEM((B,), jnp.int32),
            pltpu.VMEM((B,), jnp.float32),
            pltpu.VMEM((B,), jnp.float32),
            pltpu.SemaphoreType.DMA,
        ],
    )
    return f(t32, ac, am)


def _dense_body(a_ref, am_ref, x_ref, n_ref, o_ref, no_ref):
    n = n_ref[...]
    o_ref[...] = a_ref[...] * x_ref[...] + am_ref[...] * n
    no_ref[...] = n


def _tc_dense(a2, am2, x0T, nT):
    blk = pl.BlockSpec((PBLK, W, B), lambda i: (i, 0, 0))
    cblk = pl.BlockSpec((1, 1, B), lambda i: (0, 0, 0))
    return pl.pallas_call(
        _dense_body,
        grid=(P // PBLK,),
        in_specs=[cblk, cblk, blk, blk],
        out_specs=(blk, blk),
        out_shape=(jax.ShapeDtypeStruct((P, W, B), jnp.float32),
                   jax.ShapeDtypeStruct((P, W, B), jnp.float32)),
    )(a2, am2, x0T, nT)


@jax.jit
def _run(x0, t32, noise, ac, am):
    x0T = x0.transpose(1, 2, 3, 0).reshape(P, W, B)
    nT = noise.transpose(1, 2, 3, 0).reshape(P, W, B)
    a_all, am_all = _sc_gather(t32, ac, am)
    outT, noutT = _tc_dense(a_all.reshape(1, 1, B), am_all.reshape(1, 1, B),
                            x0T, nT)
    out = outT.reshape(C, H, W, B).transpose(3, 0, 1, 2)
    nout = noutT.reshape(C, H, W, B).transpose(3, 0, 1, 2)
    return out, nout


def kernel(x0, t, noise, sqrt_ac, sqrt_1mac):
    return _run(x0, t.astype(jnp.int32), noise, sqrt_ac, sqrt_1mac)
